# baseline (speedup 1.0000x reference)
import functools
import jax
import jax.numpy as jnp
from jax import lax
from jax.experimental import pallas as pl
from jax.experimental.pallas import tpu as pltpu
from jax.experimental.pallas import tpu_sc as plsc

_D = 64
_SCALE = 8.0

_NC = 2
_NS = 16
_NW = _NC * _NS
_L = 128
_J = 200
_JT = _J // 8


@jax.jit
def _sc_embed(xp, table):
    mesh = plsc.VectorSubcoreMesh(
        core_axis_name="c", subcore_axis_name="s", num_cores=_NC
    )

    @functools.partial(
        pl.kernel,
        mesh=mesh,
        out_type=jax.ShapeDtypeStruct((_J, 8, _NW, 8 * _L), jnp.float32),
        scratch_types=[
            pltpu.VMEM((_JT, 8 * _L), jnp.int32),
            pltpu.VMEM((2, _L, _D), jnp.float32),
            pltpu.VMEM((2, 8, 8 * _L), jnp.float32),
            pltpu.SemaphoreType.DMA,
            pltpu.SemaphoreType.DMA,
            pltpu.SemaphoreType.DMA,
            pltpu.SemaphoreType.DMA,
        ],
        compiler_params=pltpu.CompilerParams(
            use_tc_tiling_on_sc=False, needs_layout_passes=False
        ),
    )
    def k(table_hbm, xp_hbm, out_hbm, idx_slab, rows, outb,
          gsem0, gsem1, ssem0, ssem1):
        gsem = (gsem0, gsem1)
        ssem = (ssem0, ssem1)
        w = lax.axis_index("s") * _NC + lax.axis_index("c")

        pltpu.sync_copy(xp_hbm.at[:, w], idx_slab)

        iota = lax.iota(jnp.int32, 16)
        g_of_i = iota // 8
        l_base = (iota % 8) * _L

        def idx_row(j):
            return idx_slab.at[j // 8, pl.ds((j % 8) * _L, _L)]

        def gather(j, par):
            return pltpu.async_copy(
                table_hbm.at[idx_row(j)], rows.at[par], gsem[par]
            )

        gather(0, 0)

        def body(i, carry):
            for par in (0, 1):
                j = 2 * i + par
                pltpu.make_async_copy(
                    table_hbm.at[idx_row(j)], rows.at[par], gsem[par]
                ).wait()

                @pl.when(j + 1 < _J)
                def _():
                    gather(j + 1, 1 - par)

                @pl.when(j >= 2)
                def _():
                    pltpu.make_async_copy(
                        outb.at[par], out_hbm.at[j - 2, :, w], ssem[par]
                    ).wait()

                def tl(l, c):
                    lidx = l_base + l
                    for ph in range(4):
                        v = rows[par, l, pl.ds(16 * ph, 16)] * _SCALE
                        plsc.store_scatter(
                            outb.at[par], [g_of_i + 2 * ph, lidx], v
                        )
                    return c

                lax.fori_loop(0, _L, tl, 0)

                pltpu.async_copy(outb.at[par], out_hbm.at[j, :, w], ssem[par])
            return carry

        lax.fori_loop(0, _J // 2, body, 0)

        pltpu.make_async_copy(
            outb.at[0], out_hbm.at[_J - 2, :, w], ssem[0]
        ).wait()
        pltpu.make_async_copy(
            outb.at[1], out_hbm.at[_J - 1, :, w], ssem[1]
        ).wait()

    return k(table, xp)


def kernel(x, table):
    B, L = x.shape
    xp = (
        x.astype(jnp.int32)
        .T.reshape(_JT, 8, _NW, _L)
        .transpose(0, 2, 1, 3)
        .reshape(_JT, _NW, 8 * _L)
    )
    out5 = _sc_embed(xp, table).reshape(_J, 8, _NW, 8, _L)
    return out5.transpose(2, 4, 0, 1, 3).reshape(B, L, _D)


# speedup vs baseline: 1.3311x; 1.3311x over previous
import functools
import jax
import jax.numpy as jnp
from jax import lax
from jax.experimental import pallas as pl
from jax.experimental.pallas import tpu as pltpu
from jax.experimental.pallas import tpu_sc as plsc

_D = 64
_SCALE = 8.0

_NC = 2
_NS = 16
_NW = _NC * _NS
_L = 128
_J = 200
_JT = _J // 8


@jax.jit
def _sc_embed(xp, table):
    mesh = plsc.VectorSubcoreMesh(
        core_axis_name="c", subcore_axis_name="s", num_cores=_NC
    )

    @functools.partial(
        pl.kernel,
        mesh=mesh,
        out_type=jax.ShapeDtypeStruct((_J, 8, _NW, 8 * _L), jnp.float32),
        scratch_types=[
            pltpu.VMEM((_JT, 8 * _L), jnp.int32),
            pltpu.VMEM((2, _L, _D), jnp.float32),
            pltpu.VMEM((2, 8, 8 * _L), jnp.float32),
            pltpu.SemaphoreType.DMA,
            pltpu.SemaphoreType.DMA,
            pltpu.SemaphoreType.DMA,
            pltpu.SemaphoreType.DMA,
        ],
        compiler_params=pltpu.CompilerParams(
            use_tc_tiling_on_sc=False, needs_layout_passes=False
        ),
    )
    def k(table_hbm, xp_hbm, out_hbm, idx_slab, rows, outb,
          gsem0, gsem1, ssem0, ssem1):
        gsem = (gsem0, gsem1)
        ssem = (ssem0, ssem1)
        w = lax.axis_index("s") * _NC + lax.axis_index("c")

        pltpu.sync_copy(xp_hbm.at[:, w], idx_slab)

        iota = lax.iota(jnp.int32, 16)
        g_of_i = iota // 8
        l_base = (iota % 8) * _L

        def idx_row(j):
            return idx_slab.at[j // 8, pl.ds((j % 8) * _L, _L)]

        def gather(j, par):
            return pltpu.async_copy(
                table_hbm.at[idx_row(j)], rows.at[par], gsem[par]
            )

        gather(0, 0)

        def body(i, carry):
            for par in (0, 1):
                j = 2 * i + par
                pltpu.make_async_copy(
                    table_hbm.at[idx_row(j)], rows.at[par], gsem[par]
                ).wait()

                @pl.when(j + 1 < _J)
                def _():
                    gather(j + 1, 1 - par)

                @pl.when(j >= 2)
                def _():
                    pltpu.make_async_copy(
                        outb.at[par], out_hbm.at[j - 2, :, w], ssem[par]
                    ).wait()

                @plsc.parallel_loop(0, _L, unroll=8)
                def tl(l):
                    lidx = l_base + l
                    for ph in range(4):
                        v = rows[par, l, pl.ds(16 * ph, 16)] * _SCALE
                        plsc.store_scatter(
                            outb.at[par], [g_of_i + 2 * ph, lidx], v
                        )

                pltpu.async_copy(outb.at[par], out_hbm.at[j, :, w], ssem[par])
            return carry

        lax.fori_loop(0, _J // 2, body, 0)

        pltpu.make_async_copy(
            outb.at[0], out_hbm.at[_J - 2, :, w], ssem[0]
        ).wait()
        pltpu.make_async_copy(
            outb.at[1], out_hbm.at[_J - 1, :, w], ssem[1]
        ).wait()

    return k(table, xp)


def kernel(x, table):
    B, L = x.shape
    xp = (
        x.astype(jnp.int32)
        .T.reshape(_JT, 8, _NW, _L)
        .transpose(0, 2, 1, 3)
        .reshape(_JT, _NW, 8 * _L)
    )
    out5 = _sc_embed(xp, table).reshape(_J, 8, _NW, 8, _L)
    return out5.transpose(2, 4, 0, 1, 3).reshape(B, L, _D)


# speedup vs baseline: 2.0638x; 1.5504x over previous
import functools
import jax
import jax.numpy as jnp
from jax import lax
from jax.experimental import pallas as pl
from jax.experimental.pallas import tpu as pltpu
from jax.experimental.pallas import tpu_sc as plsc

_D = 64
_SCALE = 8.0

_NC = 2
_NS = 16
_NW = _NC * _NS
_L = 128
_J = 200
_JT = _J // 8


@jax.jit
def _sc_embed(xp, table):
    mesh = plsc.VectorSubcoreMesh(
        core_axis_name="c", subcore_axis_name="s", num_cores=_NC
    )

    @functools.partial(
        pl.kernel,
        mesh=mesh,
        out_type=jax.ShapeDtypeStruct((_J, 8, _NW, 8, _L), jnp.float32),
        scratch_types=[
            pltpu.VMEM((_JT, 8 * _L), jnp.int32),
            pltpu.VMEM((2, _L, _D), jnp.float32),
            pltpu.VMEM((16, 8, 129), jnp.float32),
            pltpu.SemaphoreType.DMA,
            pltpu.SemaphoreType.DMA,
            pltpu.SemaphoreType.DMA,
            pltpu.SemaphoreType.DMA,
        ],
        compiler_params=pltpu.CompilerParams(
            use_tc_tiling_on_sc=False, needs_layout_passes=False
        ),
    )
    def k(table_hbm, xp_hbm, out_hbm, idx_slab, rows, outb,
          gsem0, gsem1, ssem0, ssem1):
        gsem = (gsem0, gsem1)
        ssem = (ssem0, ssem1)
        w = lax.axis_index("s") * _NC + lax.axis_index("c")

        pltpu.sync_copy(xp_hbm.at[:, w], idx_slab)

        iota = lax.iota(jnp.int32, 16)
        g_of_i = iota // 8
        r_of_i = iota % 8

        def idx_row(j):
            return idx_slab.at[j // 8, pl.ds((j % 8) * _L, _L)]

        def gather(j, par):
            return pltpu.async_copy(
                table_hbm.at[idx_row(j)], rows.at[par], gsem[par]
            )

        gather(0, 0)

        def body(i, carry):
            for par in (0, 1):
                j = 2 * i + par
                pltpu.make_async_copy(
                    table_hbm.at[idx_row(j)], rows.at[par], gsem[par]
                ).wait()

                @pl.when(j + 1 < _J)
                def _():
                    gather(j + 1, 1 - par)

                @pl.when(j >= 2)
                def _():
                    pltpu.make_async_copy(
                        outb.at[pl.ds(par * 8, 8), :, pl.ds(0, _L)],
                        out_hbm.at[j - 2, :, w],
                        ssem[par],
                    ).wait()

                @plsc.parallel_loop(0, _L, unroll=8)
                def tl(l):
                    lvec = jnp.full((16,), l, jnp.int32)
                    for ph in range(4):
                        v = rows[par, l, pl.ds(16 * ph, 16)] * _SCALE
                        plsc.store_scatter(
                            outb,
                            [g_of_i + (par * 8 + 2 * ph), r_of_i, lvec],
                            v,
                        )

                pltpu.async_copy(
                    outb.at[pl.ds(par * 8, 8), :, pl.ds(0, _L)],
                    out_hbm.at[j, :, w],
                    ssem[par],
                )
            return carry

        lax.fori_loop(0, _J // 2, body, 0)

        pltpu.make_async_copy(
            outb.at[pl.ds(0, 8), :, pl.ds(0, _L)],
            out_hbm.at[_J - 2, :, w],
            ssem[0],
        ).wait()
        pltpu.make_async_copy(
            outb.at[pl.ds(8, 8), :, pl.ds(0, _L)],
            out_hbm.at[_J - 1, :, w],
            ssem[1],
        ).wait()

    return k(table, xp)


def kernel(x, table):
    B, L = x.shape
    xp = (
        x.astype(jnp.int32)
        .T.reshape(_JT, 8, _NW, _L)
        .transpose(0, 2, 1, 3)
        .reshape(_JT, _NW, 8 * _L)
    )
    out5 = _sc_embed(xp, table)
    return out5.transpose(2, 4, 0, 1, 3).reshape(B, L, _D)
